# Initial kernel scaffold; baseline (speedup 1.0000x reference)
import jax
import jax.numpy as jnp
from jax.experimental import pallas as pl


def kernel(x, anchor_indices, Wq, Wk, Wv, Wo):
    raise NotImplementedError("write your pallas kernel here")


# speedup vs baseline: 32.3641x; 32.3641x over previous
import functools

import jax
import jax.numpy as jnp
import numpy as np
from jax.experimental import pallas as pl
from jax.experimental.pallas import tpu as pltpu

NH = 16
DH = 64
T = 128
S = 2048
DM = 1024
KT = 4
NT = S // T
SPARSE = (KT + 1) * T


def _qkv_body(x_ref, w_ref, out_ref):
    out_ref[0, :, :] = jax.lax.dot(
        x_ref[...], w_ref[0], preferred_element_type=jnp.float32)


def _attn_one_head(anchors_ref, k_ref, v_ref, q2, base, t, a):
    cs = slice(DH * a, DH * (a + 1))
    q = q2[:, cs] * (1.0 / np.sqrt(DH))

    k_tiles = []
    v_tiles = []
    tile_ids = []
    for j in range(KT):
        idx = anchors_ref[base + j]
        k_tiles.append(k_ref[0, pl.ds(idx * T, T), :][:, cs])
        v_tiles.append(v_ref[0, pl.ds(idx * T, T), :][:, cs])
        tile_ids.append(idx)
    k_tiles.append(k_ref[0, pl.ds(t * T, T), :][:, cs])
    v_tiles.append(v_ref[0, pl.ds(t * T, T), :][:, cs])
    tile_ids.append(t)

    k_sp = jnp.concatenate(k_tiles, axis=0)
    v_sp = jnp.concatenate(v_tiles, axis=0)

    logits = jax.lax.dot_general(
        q, k_sp, (((1,), (1,)), ((), ())),
        preferred_element_type=jnp.float32)

    q_pos = t * T + jax.lax.broadcasted_iota(jnp.int32, (T, SPARSE), 0)
    off = jax.lax.broadcasted_iota(jnp.int32, (T, T), 1)
    k_pos = jnp.concatenate(
        [tid * T + off for tid in tile_ids], axis=1)
    logits = jnp.where(k_pos > q_pos, -1e10, logits)

    m = jnp.max(logits, axis=-1, keepdims=True)
    e = jnp.exp(logits - m)
    w = e / jnp.sum(e, axis=-1, keepdims=True)
    return jax.lax.dot(w, v_sp, preferred_element_type=jnp.float32)


def _attn_body(anchors_ref, q_ref, k_ref, v_ref, o_ref):
    hp = pl.program_id(0)
    t = pl.program_id(1)
    q2 = q_ref[0]
    outs = []
    for a in range(2):
        base = ((2 * hp + a) * NT + t) * KT
        outs.append(_attn_one_head(anchors_ref, k_ref, v_ref, q2, base, t, a))
    o_ref[...] = jnp.concatenate(outs, axis=1)


def _proj_body(a_ref, w_ref, out_ref):
    out_ref[...] = jax.lax.dot(
        a_ref[...], w_ref[...], preferred_element_type=jnp.float32)


@jax.jit
def kernel(x, anchor_indices, Wq, Wk, Wv, Wo):
    x2 = x.reshape(S, DM)
    w_qkv = jnp.stack([Wq, Wk, Wv])

    qkv = pl.pallas_call(
        _qkv_body,
        grid=(3,),
        in_specs=[
            pl.BlockSpec((S, DM), lambda g: (0, 0)),
            pl.BlockSpec((1, DM, NH * DH), lambda g: (g, 0, 0)),
        ],
        out_specs=pl.BlockSpec((1, S, NH * DH), lambda g: (g, 0, 0)),
        out_shape=jax.ShapeDtypeStruct((3, S, NH * DH), jnp.float32),
    )(x2, w_qkv)

    anchors_flat = anchor_indices.reshape(NH * NT * KT).astype(jnp.int32)

    attn = pl.pallas_call(
        _attn_body,
        grid_spec=pltpu.PrefetchScalarGridSpec(
            num_scalar_prefetch=1,
            grid=(NH // 2, NT),
            in_specs=[
                pl.BlockSpec((1, T, 2 * DH), lambda hp, t, a: (0, t, hp)),
                pl.BlockSpec((1, S, 2 * DH), lambda hp, t, a: (1, 0, hp)),
                pl.BlockSpec((1, S, 2 * DH), lambda hp, t, a: (2, 0, hp)),
            ],
            out_specs=pl.BlockSpec((T, 2 * DH), lambda hp, t, a: (t, hp)),
        ),
        out_shape=jax.ShapeDtypeStruct((S, NH * DH), jnp.float32),
    )(anchors_flat, qkv, qkv, qkv)

    out = pl.pallas_call(
        _proj_body,
        in_specs=[
            pl.BlockSpec((S, NH * DH), lambda: (0, 0)),
            pl.BlockSpec((NH * DH, DM), lambda: (0, 0)),
        ],
        out_specs=pl.BlockSpec((S, DM), lambda: (0, 0)),
        out_shape=jax.ShapeDtypeStruct((S, DM), jnp.float32),
        grid=(),
    )(attn, Wo)

    return out.reshape(1, S, DM)


# speedup vs baseline: 36.1121x; 1.1158x over previous
import functools

import jax
import jax.numpy as jnp
import numpy as np
from jax.experimental import pallas as pl
from jax.experimental.pallas import tpu as pltpu

NH = 16
DH = 64
T = 128
S = 2048
DM = 1024
KT = 4
NT = S // T
SPARSE = (KT + 1) * T


def _qkv_body(x_ref, w_ref, out_ref):
    out_ref[0, :, :] = jax.lax.dot(
        x_ref[...], w_ref[0], preferred_element_type=jnp.float32)


def _attn_body(anchors_ref, q_ref, k_ref, v_ref, o_ref):
    hp = pl.program_id(0)
    t = pl.program_id(1)
    q2 = q_ref[0] * (1.0 / np.sqrt(DH))
    col = jax.lax.broadcasted_iota(jnp.int32, (T, 2 * DH), 1)
    m0 = (col < DH).astype(jnp.float32)
    qh = [q2 * m0, q2 * (1.0 - m0)]

    tiles = []
    base0 = ((2 * hp + 0) * NT + t) * KT
    base1 = ((2 * hp + 1) * NT + t) * KT
    ids = [[anchors_ref[base0 + j] for j in range(KT)] + [t],
           [anchors_ref[base1 + j] for j in range(KT)] + [t]]

    q_pos = t * T + jax.lax.broadcasted_iota(jnp.int32, (T, T), 0)
    k_off = jax.lax.broadcasted_iota(jnp.int32, (T, T), 1)

    outs = []
    for a in range(2):
        logit_tiles = []
        v_tiles = []
        for idx in ids[a]:
            kj = k_ref[0, pl.ds(idx * T, T), :]
            v_tiles.append(v_ref[0, pl.ds(idx * T, T), :])
            l = jax.lax.dot_general(
                qh[a], kj, (((1,), (1,)), ((), ())),
                preferred_element_type=jnp.float32)
            l = jnp.where(idx * T + k_off > q_pos, -1e10, l)
            logit_tiles.append(l)
        logits = jnp.concatenate(logit_tiles, axis=1)
        m = jnp.max(logits, axis=-1, keepdims=True)
        e = jnp.exp(logits - m)
        s = jnp.sum(e, axis=-1, keepdims=True)
        acc = jnp.zeros((T, 2 * DH), jnp.float32)
        for j in range(KT + 1):
            w = e[:, j * T:(j + 1) * T]
            acc = acc + jax.lax.dot(
                w, v_tiles[j], preferred_element_type=jnp.float32)
        outs.append(acc / s)
    o_ref[...] = outs[0] * m0 + outs[1] * (1.0 - m0)


def _proj_body(a_ref, w_ref, out_ref):
    out_ref[...] = jax.lax.dot(
        a_ref[...], w_ref[...], preferred_element_type=jnp.float32)


@jax.jit
def kernel(x, anchor_indices, Wq, Wk, Wv, Wo):
    x2 = x.reshape(S, DM)
    w_qkv = jnp.stack([Wq, Wk, Wv])

    qkv = pl.pallas_call(
        _qkv_body,
        grid=(3,),
        in_specs=[
            pl.BlockSpec((S, DM), lambda g: (0, 0)),
            pl.BlockSpec((1, DM, NH * DH), lambda g: (g, 0, 0)),
        ],
        out_specs=pl.BlockSpec((1, S, NH * DH), lambda g: (g, 0, 0)),
        out_shape=jax.ShapeDtypeStruct((3, S, NH * DH), jnp.float32),
    )(x2, w_qkv)

    anchors_flat = anchor_indices.reshape(NH * NT * KT).astype(jnp.int32)

    attn = pl.pallas_call(
        _attn_body,
        grid_spec=pltpu.PrefetchScalarGridSpec(
            num_scalar_prefetch=1,
            grid=(NH // 2, NT),
            in_specs=[
                pl.BlockSpec((1, T, 2 * DH), lambda hp, t, a: (0, t, hp)),
                pl.BlockSpec((1, S, 2 * DH), lambda hp, t, a: (1, 0, hp)),
                pl.BlockSpec((1, S, 2 * DH), lambda hp, t, a: (2, 0, hp)),
            ],
            out_specs=pl.BlockSpec((T, 2 * DH), lambda hp, t, a: (t, hp)),
        ),
        out_shape=jax.ShapeDtypeStruct((S, NH * DH), jnp.float32),
    )(anchors_flat, qkv, qkv, qkv)

    out = pl.pallas_call(
        _proj_body,
        in_specs=[
            pl.BlockSpec((S, NH * DH), lambda: (0, 0)),
            pl.BlockSpec((NH * DH, DM), lambda: (0, 0)),
        ],
        out_specs=pl.BlockSpec((S, DM), lambda: (0, 0)),
        out_shape=jax.ShapeDtypeStruct((S, DM), jnp.float32),
        grid=(),
    )(attn, Wo)

    return out.reshape(1, S, DM)
